# Initial kernel scaffold; baseline (speedup 1.0000x reference)
import jax
import jax.numpy as jnp
from jax.experimental import pallas as pl


def kernel(x, edge_index, W_in, b_in, ln1_g, ln1_b, Wg1, bg1, Wg2, bg2, ln2_g, ln2_b, Wg3, bg3, Wqkv, bqkv, Wo, bo, Wc1, bc1, Wc2, bc2, Wc3, bc3):
    raise NotImplementedError("write your pallas kernel here")


# speedup vs baseline: 14.4657x; 14.4657x over previous
import functools

import jax
import jax.numpy as jnp
from jax import lax
from jax.experimental import pallas as pl
from jax.experimental.pallas import tpu as pltpu
from jax.experimental.pallas import tpu_sc as plsc

N = 50000
E = 800000
LANES = 128
EROWS = 6400
EPAD = EROWS * LANES
NACC = 50048
DUMP = N
CH = 40
ZROWS = 184
RB = 2000
GRID = N // RB


def _fill2d(ref, rows, width, value):
    v = jnp.full((16,), value, dtype=jnp.float32)

    def body(i, _):
        for f in range(width // 16):
            ref[i, pl.ds(f * 16, 16)] = v
        return 0

    lax.fori_loop(0, rows, body, 0)


def _sc_mesh():
    return plsc.VectorSubcoreMesh(core_axis_name="c", subcore_axis_name="s")


_SC_PARAMS = pltpu.CompilerParams(use_tc_tiling_on_sc=False)


def _make_deg_kernel():
    @functools.partial(
        pl.kernel,
        out_type=jax.ShapeDtypeStruct((2, NACC, 16), jnp.float32),
        mesh=_sc_mesh(),
        compiler_params=_SC_PARAMS,
        scratch_types=[
            pltpu.VMEM((CH, LANES), jnp.int32),
            pltpu.VMEM((LANES, 16), jnp.float32),
            pltpu.VMEM((ZROWS, 16), jnp.float32),
            pltpu.VMEM_SHARED((NACC, 16), jnp.float32),
        ],
    )
    def k(dstp, out, didx, ones_v, zbuf, acc):
        c = lax.axis_index("c")
        s = lax.axis_index("s")
        _fill2d(ones_v, LANES, 16, 1.0)
        _fill2d(zbuf, ZROWS, 16, 0.0)
        zbase = s * (NACC // 16)
        for z in range(17):
            pltpu.sync_copy(zbuf, acc.at[pl.ds(zbase + z * ZROWS, ZROWS)])
        plsc.subcore_barrier()

        rows_per_tile = EROWS // 32
        row_base = (c * 16 + s) * rows_per_tile

        def outer(ob, _):
            rb = row_base + ob * CH
            pltpu.sync_copy(dstp.at[pl.ds(rb, CH)], didx)

            def inner(j, _):
                pltpu.sync_copy(ones_v, acc.at[didx.at[j]], add=True)
                return 0

            lax.fori_loop(0, CH, inner, 0)
            return 0

        lax.fori_loop(0, rows_per_tile // CH, outer, 0)
        plsc.subcore_barrier()
        nout = NACC // 16
        pltpu.sync_copy(acc.at[pl.ds(s * nout, nout)],
                        out.at[c, pl.ds(s * nout, nout)])

    return k


def _make_agg_kernel(f2, colsplit):
    n_in = 2 if colsplit else 1
    rows_per_tile = EROWS // 16 if colsplit else EROWS // 32

    def body(*refs):
        tables = refs[:n_in]
        srcp, dstp = refs[n_in], refs[n_in + 1]
        out = refs[n_in + 2]
        sidx, didx, rows_v, zbuf, acc, sem = refs[n_in + 3:]
        c = lax.axis_index("c")
        s = lax.axis_index("s")
        _fill2d(zbuf, ZROWS, f2, 0.0)
        zbase = s * (NACC // 16)
        for z in range(17):
            pltpu.sync_copy(zbuf, acc.at[pl.ds(zbase + z * ZROWS, ZROWS)])
        plsc.subcore_barrier()

        if colsplit:
            row_base = s * rows_per_tile
        else:
            row_base = (c * 16 + s) * rows_per_tile

        def outer(ob, _):
            rb = row_base + ob * CH
            pltpu.sync_copy(srcp.at[pl.ds(rb, CH)], sidx)
            pltpu.sync_copy(dstp.at[pl.ds(rb, CH)], didx)

            def inner(j, _):
                if colsplit:
                    @pl.when(c == 0)
                    def _():
                        pltpu.async_copy(tables[0].at[sidx.at[j]], rows_v,
                                         sem).wait()

                    @pl.when(c == 1)
                    def _():
                        pltpu.async_copy(tables[1].at[sidx.at[j]], rows_v,
                                         sem).wait()
                else:
                    pltpu.async_copy(tables[0].at[sidx.at[j]], rows_v,
                                     sem).wait()
                pltpu.sync_copy(rows_v, acc.at[didx.at[j]], add=True)
                return 0

            lax.fori_loop(0, CH, inner, 0)
            return 0

        lax.fori_loop(0, rows_per_tile // CH, outer, 0)
        plsc.subcore_barrier()
        nout = NACC // 16
        pltpu.sync_copy(acc.at[pl.ds(s * nout, nout)],
                        out.at[c, pl.ds(s * nout, nout)])

    return functools.partial(
        pl.kernel,
        out_type=jax.ShapeDtypeStruct((2, NACC, f2), jnp.float32),
        mesh=_sc_mesh(),
        compiler_params=_SC_PARAMS,
        scratch_types=[
            pltpu.VMEM((CH, LANES), jnp.int32),
            pltpu.VMEM((CH, LANES), jnp.int32),
            pltpu.VMEM((LANES, f2), jnp.float32),
            pltpu.VMEM((ZROWS, f2), jnp.float32),
            pltpu.VMEM_SHARED((NACC, f2), jnp.float32),
            pltpu.SemaphoreType.DMA,
        ],
    )(body)


_deg_call = _make_deg_kernel()
_agg64 = _make_agg_kernel(32, colsplit=True)
_agg32 = _make_agg_kernel(32, colsplit=False)
_agg16 = _make_agg_kernel(16, colsplit=False)


def _full(shape):
    return pl.BlockSpec(shape, lambda i: (0,) * len(shape))


def _rows(shape):
    nd = len(shape)
    if nd == 2:
        return pl.BlockSpec(shape, lambda i: (i, 0))
    return pl.BlockSpec(shape, lambda i: (0, i, 0))


def _tc1_body(x_ref, degp_ref, wint_ref, bin_ref, g1_ref, b1_ref, wg1t_ref,
              h_ref, hs1a_ref, hs1b_ref, dinv_ref):
    xb = x_ref[...]
    h0 = jnp.maximum(
        jnp.dot(xb, wint_ref[...], preferred_element_type=jnp.float32)
        + bin_ref[...], 0.0)
    m = jnp.mean(h0, axis=-1, keepdims=True)
    v = jnp.mean((h0 - m) ** 2, axis=-1, keepdims=True)
    hb = (h0 - m) / jnp.sqrt(v + 1e-5) * g1_ref[...] + b1_ref[...]
    deg = degp_ref[0, :, 0] + degp_ref[1, :, 0] + 1.0
    dinv = lax.rsqrt(deg).reshape(RB, 1)
    hl1 = jnp.dot(hb, wg1t_ref[...], preferred_element_type=jnp.float32)
    hs1 = hl1 * dinv
    h_ref[...] = hb
    hs1a_ref[...] = hs1[:, :32]
    hs1b_ref[...] = hs1[:, 32:]
    dinv_ref[...] = dinv


def _tc1(xp, degp, wint, bin_, g1, b1, wg1t):
    return pl.pallas_call(
        _tc1_body,
        grid=(GRID,),
        in_specs=[
            _rows((RB, 192)),
            _rows((2, RB, 16)),
            _full((192, 64)),
            _full((1, 64)),
            _full((1, 64)),
            _full((1, 64)),
            _full((64, 64)),
        ],
        out_specs=[
            _rows((RB, 64)),
            _rows((RB, 32)),
            _rows((RB, 32)),
            _rows((RB, 1)),
        ],
        out_shape=[
            jax.ShapeDtypeStruct((N, 64), jnp.float32),
            jax.ShapeDtypeStruct((N, 32), jnp.float32),
            jax.ShapeDtypeStruct((N, 32), jnp.float32),
            jax.ShapeDtypeStruct((N, 1), jnp.float32),
        ],
    )(xp, degp, wint, bin_, g1, b1, wg1t)


def _tc2_body(h_ref, a1_ref, hs1a_ref, hs1b_ref, dinv_ref, bg1_ref, wg2t_ref,
              hs2_ref):
    dinv = dinv_ref[...]
    left = a1_ref[0] + hs1a_ref[...]
    right = a1_ref[1] + hs1b_ref[...]
    agg = jnp.concatenate([left, right], axis=1)
    t = jnp.maximum(agg * dinv + bg1_ref[...], 0.0)
    h1 = t + h_ref[...]
    hs2_ref[...] = jnp.dot(h1, wg2t_ref[...],
                           preferred_element_type=jnp.float32) * dinv


def _tc2(h, a1, hs1a, hs1b, dinv, bg1, wg2t):
    return pl.pallas_call(
        _tc2_body,
        grid=(GRID,),
        in_specs=[
            _rows((RB, 64)),
            _rows((2, RB, 32)),
            _rows((RB, 32)),
            _rows((RB, 32)),
            _rows((RB, 1)),
            _full((1, 64)),
            _full((64, 32)),
        ],
        out_specs=[_rows((RB, 32))],
        out_shape=[jax.ShapeDtypeStruct((N, 32), jnp.float32)],
    )(h, a1, hs1a, hs1b, dinv, bg1, wg2t)[0]


def _tc3_body(a2_ref, hs2_ref, dinv_ref, bg2_ref, g2_ref, b2_ref, wg3t_ref,
              hs3_ref):
    dinv = dinv_ref[...]
    agg = a2_ref[0] + a2_ref[1] + hs2_ref[...]
    t = jnp.maximum(agg * dinv + bg2_ref[...], 0.0)
    m = jnp.mean(t, axis=-1, keepdims=True)
    v = jnp.mean((t - m) ** 2, axis=-1, keepdims=True)
    h2 = (t - m) / jnp.sqrt(v + 1e-5) * g2_ref[...] + b2_ref[...]
    hs3_ref[...] = jnp.dot(h2, wg3t_ref[...],
                           preferred_element_type=jnp.float32) * dinv


def _tc3(a2, hs2, dinv, bg2, g2, b2, wg3t):
    return pl.pallas_call(
        _tc3_body,
        grid=(GRID,),
        in_specs=[
            _rows((2, RB, 32)),
            _rows((RB, 32)),
            _rows((RB, 1)),
            _full((1, 32)),
            _full((1, 32)),
            _full((1, 32)),
            _full((32, 16)),
        ],
        out_specs=[_rows((RB, 16))],
        out_shape=[jax.ShapeDtypeStruct((N, 16), jnp.float32)],
    )(a2, hs2, dinv, bg2, g2, b2, wg3t)[0]


def _tc4_body(a3_ref, hs3_ref, dinv_ref, bg3_ref, wvt_ref, bv_ref, wot_ref,
              bo_ref, wc1t_ref, bc1_ref, wc2t_ref, bc2_ref, wc3t_ref,
              bc3_ref, out_ref):
    dinv = dinv_ref[...]
    agg = a3_ref[0] + a3_ref[1] + hs3_ref[...]
    h3 = jnp.maximum(agg * dinv + bg3_ref[...], 0.0)
    vv = jnp.dot(h3, wvt_ref[...], preferred_element_type=jnp.float32) \
        + bv_ref[...]
    att = jnp.dot(vv, wot_ref[...], preferred_element_type=jnp.float32) \
        + bo_ref[...]
    p = jnp.maximum(
        jnp.dot(att, wc1t_ref[...], preferred_element_type=jnp.float32)
        + bc1_ref[...], 0.0)
    p = jnp.maximum(
        jnp.dot(p, wc2t_ref[...], preferred_element_type=jnp.float32)
        + bc2_ref[...], 0.0)
    out_ref[...] = jnp.dot(p, wc3t_ref[...],
                           preferred_element_type=jnp.float32) + bc3_ref[...]


def _tc4(a3, hs3, dinv, bg3, wvt, bv, wot, bo, wc1t, bc1, wc2t, bc2, wc3t,
         bc3):
    return pl.pallas_call(
        _tc4_body,
        grid=(GRID,),
        in_specs=[
            _rows((2, RB, 16)),
            _rows((RB, 16)),
            _rows((RB, 1)),
            _full((1, 16)),
            _full((16, 16)),
            _full((1, 16)),
            _full((16, 16)),
            _full((1, 16)),
            _full((16, 8)),
            _full((1, 8)),
            _full((8, 32)),
            _full((1, 32)),
            _full((32, 1)),
            _full((1, 1)),
        ],
        out_specs=[_rows((RB, 1))],
        out_shape=[jax.ShapeDtypeStruct((N, 1), jnp.float32)],
    )(a3, hs3, dinv, bg3, wvt, bv, wot, bo, wc1t, bc1, wc2t, bc2, wc3t, bc3)[0]


def kernel(x, edge_index, W_in, b_in, ln1_g, ln1_b, Wg1, bg1, Wg2, bg2,
           ln2_g, ln2_b, Wg3, bg3, Wqkv, bqkv, Wo, bo, Wc1, bc1, Wc2, bc2,
           Wc3, bc3):
    src = edge_index[0]
    dst = edge_index[1]
    srcp = jnp.pad(src, (0, EPAD - E)).reshape(EROWS, LANES)
    dstp = jnp.pad(dst, (0, EPAD - E),
                   constant_values=DUMP).reshape(EROWS, LANES)

    xp = jnp.pad(x, ((0, 0), (0, 192 - x.shape[1])))
    wint = jnp.pad(W_in.T, ((0, 192 - W_in.shape[1]), (0, 0)))

    degp = _deg_call(dstp)

    h, hs1a, hs1b, dinv = _tc1(
        xp, degp, wint, b_in.reshape(1, 64), ln1_g.reshape(1, 64),
        ln1_b.reshape(1, 64), Wg1.T)

    a1 = _agg64(hs1a, hs1b, srcp, dstp)

    hs2 = _tc2(h, a1, hs1a, hs1b, dinv, bg1.reshape(1, 64), Wg2.T)

    a2 = _agg32(hs2, srcp, dstp)

    hs3 = _tc3(a2, hs2, dinv, bg2.reshape(1, 32), ln2_g.reshape(1, 32),
               ln2_b.reshape(1, 32), Wg3.T)

    a3 = _agg16(hs3, srcp, dstp)

    wv = Wqkv[32:48]
    bv = bqkv[32:48]
    out = _tc4(a3, hs3, dinv, bg3.reshape(1, 16), wv.T, bv.reshape(1, 16),
               Wo.T, bo.reshape(1, 16), Wc1.T, bc1.reshape(1, 8), Wc2.T,
               bc2.reshape(1, 32), Wc3.T, bc3.reshape(1, 1))
    return out[:, 0]


# speedup vs baseline: 16.9667x; 1.1729x over previous
import functools

import jax
import jax.numpy as jnp
from jax import lax
from jax.experimental import pallas as pl
from jax.experimental.pallas import tpu as pltpu
from jax.experimental.pallas import tpu_sc as plsc

N = 50000
E = 800000
LANES = 128
EROWS = 6400
EPAD = EROWS * LANES
NACC = 50048
DUMP = N
CH = 40
ZROWS = 184
RB = 2000
GRID = N // RB


def _fill2d(ref, rows, width, value):
    v = jnp.full((16,), value, dtype=jnp.float32)

    def body(i, _):
        for f in range(width // 16):
            ref[i, pl.ds(f * 16, 16)] = v
        return 0

    lax.fori_loop(0, rows, body, 0)


def _sc_mesh():
    return plsc.VectorSubcoreMesh(core_axis_name="c", subcore_axis_name="s")


_SC_PARAMS = pltpu.CompilerParams(use_tc_tiling_on_sc=False)


def _make_deg_kernel():
    @functools.partial(
        pl.kernel,
        out_type=jax.ShapeDtypeStruct((2, NACC, 16), jnp.float32),
        mesh=_sc_mesh(),
        compiler_params=_SC_PARAMS,
        scratch_types=[
            pltpu.VMEM((CH, LANES), jnp.int32),
            pltpu.VMEM((LANES, 16), jnp.float32),
            pltpu.VMEM((ZROWS, 16), jnp.float32),
            pltpu.VMEM_SHARED((NACC, 16), jnp.float32),
        ],
    )
    def k(dstp, out, didx, ones_v, zbuf, acc):
        c = lax.axis_index("c")
        s = lax.axis_index("s")
        _fill2d(ones_v, LANES, 16, 1.0)
        _fill2d(zbuf, ZROWS, 16, 0.0)
        zbase = s * (NACC // 16)
        for z in range(17):
            pltpu.sync_copy(zbuf, acc.at[pl.ds(zbase + z * ZROWS, ZROWS)])
        plsc.subcore_barrier()

        rows_per_tile = EROWS // 32
        row_base = (c * 16 + s) * rows_per_tile

        def outer(ob, _):
            rb = row_base + ob * CH
            pltpu.sync_copy(dstp.at[pl.ds(rb, CH)], didx)

            def inner(j, _):
                pltpu.sync_copy(ones_v, acc.at[didx.at[j]], add=True)
                return 0

            lax.fori_loop(0, CH, inner, 0)
            return 0

        lax.fori_loop(0, rows_per_tile // CH, outer, 0)
        plsc.subcore_barrier()
        nout = NACC // 16
        pltpu.sync_copy(acc.at[pl.ds(s * nout, nout)],
                        out.at[c, pl.ds(s * nout, nout)])

    return k


def _make_agg_kernel(f2, colsplit):
    n_in = 2 if colsplit else 1
    rows_per_tile = EROWS // 16 if colsplit else EROWS // 32

    def body(*refs):
        tables = refs[:n_in]
        srcp, dstp = refs[n_in], refs[n_in + 1]
        out = refs[n_in + 2]
        (sidx, didx, rows_a, rows_b, zbuf, acc,
         gsem_a, gsem_b, ssem_a, ssem_b) = refs[n_in + 3:]
        c = lax.axis_index("c")
        s = lax.axis_index("s")
        _fill2d(zbuf, ZROWS, f2, 0.0)
        zbase = s * (NACC // 16)
        for z in range(17):
            pltpu.sync_copy(zbuf, acc.at[pl.ds(zbase + z * ZROWS, ZROWS)])
        plsc.subcore_barrier()

        if colsplit:
            row_base = s * rows_per_tile
        else:
            row_base = (c * 16 + s) * rows_per_tile

        bufs = (rows_a, rows_b)
        gsems = (gsem_a, gsem_b)
        ssems = (ssem_a, ssem_b)

        def gather(j, k):
            if colsplit:
                @pl.when(c == 0)
                def _():
                    pltpu.async_copy(tables[0].at[sidx.at[j]], bufs[k],
                                     gsems[k])

                @pl.when(c == 1)
                def _():
                    pltpu.async_copy(tables[1].at[sidx.at[j]], bufs[k],
                                     gsems[k])
            else:
                pltpu.async_copy(tables[0].at[sidx.at[j]], bufs[k], gsems[k])

        def gwait(j, k):
            pltpu.make_async_copy(tables[0].at[sidx.at[j]], bufs[k],
                                  gsems[k]).wait()

        def scat(j, k):
            pltpu.async_copy(bufs[k], acc.at[didx.at[j]], ssems[k], add=True)

        def swait(j, k):
            pltpu.make_async_copy(bufs[k], acc.at[didx.at[j]],
                                  ssems[k]).wait()

        def outer(ob, _):
            rb = row_base + ob * CH
            pltpu.sync_copy(srcp.at[pl.ds(rb, CH)], sidx)
            pltpu.sync_copy(dstp.at[pl.ds(rb, CH)], didx)
            gather(0, 0)
            gather(1, 1)

            def inner2(j2, _):
                j0 = 2 * j2
                gwait(j0, 0)
                scat(j0, 0)
                gwait(j0 + 1, 1)
                scat(j0 + 1, 1)

                @pl.when(j2 + 1 < CH // 2)
                def _():
                    swait(j0, 0)
                    gather(j0 + 2, 0)
                    swait(j0 + 1, 1)
                    gather(j0 + 3, 1)
                return 0

            lax.fori_loop(0, CH // 2, inner2, 0)
            swait(CH - 2, 0)
            swait(CH - 1, 1)
            return 0

        lax.fori_loop(0, rows_per_tile // CH, outer, 0)
        plsc.subcore_barrier()
        nout = NACC // 16
        pltpu.sync_copy(acc.at[pl.ds(s * nout, nout)],
                        out.at[c, pl.ds(s * nout, nout)])

    return functools.partial(
        pl.kernel,
        out_type=jax.ShapeDtypeStruct((2, NACC, f2), jnp.float32),
        mesh=_sc_mesh(),
        compiler_params=_SC_PARAMS,
        scratch_types=[
            pltpu.VMEM((CH, LANES), jnp.int32),
            pltpu.VMEM((CH, LANES), jnp.int32),
            pltpu.VMEM((LANES, f2), jnp.float32),
            pltpu.VMEM((LANES, f2), jnp.float32),
            pltpu.VMEM((ZROWS, f2), jnp.float32),
            pltpu.VMEM_SHARED((NACC, f2), jnp.float32),
            pltpu.SemaphoreType.DMA,
            pltpu.SemaphoreType.DMA,
            pltpu.SemaphoreType.DMA,
            pltpu.SemaphoreType.DMA,
        ],
    )(body)


_deg_call = _make_deg_kernel()
_agg64 = _make_agg_kernel(32, colsplit=True)
_agg32 = _make_agg_kernel(32, colsplit=False)
_agg16 = _make_agg_kernel(16, colsplit=False)


def _full(shape):
    return pl.BlockSpec(shape, lambda i: (0,) * len(shape))


def _rows(shape):
    nd = len(shape)
    if nd == 2:
        return pl.BlockSpec(shape, lambda i: (i, 0))
    return pl.BlockSpec(shape, lambda i: (0, i, 0))


def _tc1_body(x_ref, degp_ref, wint_ref, bin_ref, g1_ref, b1_ref, wg1t_ref,
              h_ref, hs1a_ref, hs1b_ref, dinv_ref):
    xb = x_ref[...]
    h0 = jnp.maximum(
        jnp.dot(xb, wint_ref[...], preferred_element_type=jnp.float32)
        + bin_ref[...], 0.0)
    m = jnp.mean(h0, axis=-1, keepdims=True)
    v = jnp.mean((h0 - m) ** 2, axis=-1, keepdims=True)
    hb = (h0 - m) / jnp.sqrt(v + 1e-5) * g1_ref[...] + b1_ref[...]
    deg = degp_ref[0, :, 0] + degp_ref[1, :, 0] + 1.0
    dinv = lax.rsqrt(deg).reshape(RB, 1)
    hl1 = jnp.dot(hb, wg1t_ref[...], preferred_element_type=jnp.float32)
    hs1 = hl1 * dinv
    h_ref[...] = hb
    hs1a_ref[...] = hs1[:, :32]
    hs1b_ref[...] = hs1[:, 32:]
    dinv_ref[...] = dinv


def _tc1(xp, degp, wint, bin_, g1, b1, wg1t):
    return pl.pallas_call(
        _tc1_body,
        grid=(GRID,),
        in_specs=[
            _rows((RB, 192)),
            _rows((2, RB, 16)),
            _full((192, 64)),
            _full((1, 64)),
            _full((1, 64)),
            _full((1, 64)),
            _full((64, 64)),
        ],
        out_specs=[
            _rows((RB, 64)),
            _rows((RB, 32)),
            _rows((RB, 32)),
            _rows((RB, 1)),
        ],
        out_shape=[
            jax.ShapeDtypeStruct((N, 64), jnp.float32),
            jax.ShapeDtypeStruct((N, 32), jnp.float32),
            jax.ShapeDtypeStruct((N, 32), jnp.float32),
            jax.ShapeDtypeStruct((N, 1), jnp.float32),
        ],
    )(xp, degp, wint, bin_, g1, b1, wg1t)


def _tc2_body(h_ref, a1_ref, hs1a_ref, hs1b_ref, dinv_ref, bg1_ref, wg2t_ref,
              hs2_ref):
    dinv = dinv_ref[...]
    left = a1_ref[0] + hs1a_ref[...]
    right = a1_ref[1] + hs1b_ref[...]
    agg = jnp.concatenate([left, right], axis=1)
    t = jnp.maximum(agg * dinv + bg1_ref[...], 0.0)
    h1 = t + h_ref[...]
    hs2_ref[...] = jnp.dot(h1, wg2t_ref[...],
                           preferred_element_type=jnp.float32) * dinv


def _tc2(h, a1, hs1a, hs1b, dinv, bg1, wg2t):
    return pl.pallas_call(
        _tc2_body,
        grid=(GRID,),
        in_specs=[
            _rows((RB, 64)),
            _rows((2, RB, 32)),
            _rows((RB, 32)),
            _rows((RB, 32)),
            _rows((RB, 1)),
            _full((1, 64)),
            _full((64, 32)),
        ],
        out_specs=[_rows((RB, 32))],
        out_shape=[jax.ShapeDtypeStruct((N, 32), jnp.float32)],
    )(h, a1, hs1a, hs1b, dinv, bg1, wg2t)[0]


def _tc3_body(a2_ref, hs2_ref, dinv_ref, bg2_ref, g2_ref, b2_ref, wg3t_ref,
              hs3_ref):
    dinv = dinv_ref[...]
    agg = a2_ref[0] + a2_ref[1] + hs2_ref[...]
    t = jnp.maximum(agg * dinv + bg2_ref[...], 0.0)
    m = jnp.mean(t, axis=-1, keepdims=True)
    v = jnp.mean((t - m) ** 2, axis=-1, keepdims=True)
    h2 = (t - m) / jnp.sqrt(v + 1e-5) * g2_ref[...] + b2_ref[...]
    hs3_ref[...] = jnp.dot(h2, wg3t_ref[...],
                           preferred_element_type=jnp.float32) * dinv


def _tc3(a2, hs2, dinv, bg2, g2, b2, wg3t):
    return pl.pallas_call(
        _tc3_body,
        grid=(GRID,),
        in_specs=[
            _rows((2, RB, 32)),
            _rows((RB, 32)),
            _rows((RB, 1)),
            _full((1, 32)),
            _full((1, 32)),
            _full((1, 32)),
            _full((32, 16)),
        ],
        out_specs=[_rows((RB, 16))],
        out_shape=[jax.ShapeDtypeStruct((N, 16), jnp.float32)],
    )(a2, hs2, dinv, bg2, g2, b2, wg3t)[0]


def _tc4_body(a3_ref, hs3_ref, dinv_ref, bg3_ref, wvt_ref, bv_ref, wot_ref,
              bo_ref, wc1t_ref, bc1_ref, wc2t_ref, bc2_ref, wc3t_ref,
              bc3_ref, out_ref):
    dinv = dinv_ref[...]
    agg = a3_ref[0] + a3_ref[1] + hs3_ref[...]
    h3 = jnp.maximum(agg * dinv + bg3_ref[...], 0.0)
    vv = jnp.dot(h3, wvt_ref[...], preferred_element_type=jnp.float32) \
        + bv_ref[...]
    att = jnp.dot(vv, wot_ref[...], preferred_element_type=jnp.float32) \
        + bo_ref[...]
    p = jnp.maximum(
        jnp.dot(att, wc1t_ref[...], preferred_element_type=jnp.float32)
        + bc1_ref[...], 0.0)
    p = jnp.maximum(
        jnp.dot(p, wc2t_ref[...], preferred_element_type=jnp.float32)
        + bc2_ref[...], 0.0)
    out_ref[...] = jnp.dot(p, wc3t_ref[...],
                           preferred_element_type=jnp.float32) + bc3_ref[...]


def _tc4(a3, hs3, dinv, bg3, wvt, bv, wot, bo, wc1t, bc1, wc2t, bc2, wc3t,
         bc3):
    return pl.pallas_call(
        _tc4_body,
        grid=(GRID,),
        in_specs=[
            _rows((2, RB, 16)),
            _rows((RB, 16)),
            _rows((RB, 1)),
            _full((1, 16)),
            _full((16, 16)),
            _full((1, 16)),
            _full((16, 16)),
            _full((1, 16)),
            _full((16, 8)),
            _full((1, 8)),
            _full((8, 32)),
            _full((1, 32)),
            _full((32, 1)),
            _full((1, 1)),
        ],
        out_specs=[_rows((RB, 1))],
        out_shape=[jax.ShapeDtypeStruct((N, 1), jnp.float32)],
    )(a3, hs3, dinv, bg3, wvt, bv, wot, bo, wc1t, bc1, wc2t, bc2, wc3t, bc3)[0]


def kernel(x, edge_index, W_in, b_in, ln1_g, ln1_b, Wg1, bg1, Wg2, bg2,
           ln2_g, ln2_b, Wg3, bg3, Wqkv, bqkv, Wo, bo, Wc1, bc1, Wc2, bc2,
           Wc3, bc3):
    src = edge_index[0]
    dst = edge_index[1]
    srcp = jnp.pad(src, (0, EPAD - E)).reshape(EROWS, LANES)
    dstp = jnp.pad(dst, (0, EPAD - E),
                   constant_values=DUMP).reshape(EROWS, LANES)

    xp = jnp.pad(x, ((0, 0), (0, 192 - x.shape[1])))
    wint = jnp.pad(W_in.T, ((0, 192 - W_in.shape[1]), (0, 0)))

    degp = _deg_call(dstp)

    h, hs1a, hs1b, dinv = _tc1(
        xp, degp, wint, b_in.reshape(1, 64), ln1_g.reshape(1, 64),
        ln1_b.reshape(1, 64), Wg1.T)

    a1 = _agg64(hs1a, hs1b, srcp, dstp)

    hs2 = _tc2(h, a1, hs1a, hs1b, dinv, bg1.reshape(1, 64), Wg2.T)

    a2 = _agg32(hs2, srcp, dstp)

    hs3 = _tc3(a2, hs2, dinv, bg2.reshape(1, 32), ln2_g.reshape(1, 32),
               ln2_b.reshape(1, 32), Wg3.T)

    a3 = _agg16(hs3, srcp, dstp)

    wv = Wqkv[32:48]
    bv = bqkv[32:48]
    out = _tc4(a3, hs3, dinv, bg3.reshape(1, 16), wv.T, bv.reshape(1, 16),
               Wo.T, bo.reshape(1, 16), Wc1.T, bc1.reshape(1, 8), Wc2.T,
               bc2.reshape(1, 32), Wc3.T, bc3.reshape(1, 1))
    return out[:, 0]
